# baseline (speedup 1.0000x reference)
import functools

import jax
import jax.numpy as jnp
from jax import lax
from jax.experimental import pallas as pl
from jax.experimental.pallas import tpu as pltpu
from jax.experimental.pallas import tpu_sc as plsc

_NC = 2
_NS = 16
_NW = _NC * _NS
_L = 16
_K = 80


def _interp_sc(tbl, pxs, pys, B, C, Hx, Wx, N):
    P = B * N
    n_chunks = P // _K
    cpb = N // _K
    base = n_chunks // _NW
    rem = n_chunks % _NW
    maxw = base + 1

    mesh = plsc.VectorSubcoreMesh(
        core_axis_name="c", subcore_axis_name="s",
        num_cores=_NC, num_subcores=_NS)

    @functools.partial(
        pl.kernel,
        out_type=jax.ShapeDtypeStruct((B, N, C), jnp.float32),
        mesh=mesh,
        scratch_types=[
            pltpu.VMEM((maxw * _K,), jnp.float32),
            pltpu.VMEM((maxw * _K,), jnp.float32),
            pltpu.VMEM((2, 4, _K), jnp.int32),
            pltpu.VMEM((2, 4, _K), jnp.float32),
            pltpu.VMEM((2, 4, _K, C), jnp.bfloat16),
            pltpu.VMEM((2, _K, C), jnp.float32),
            pltpu.SemaphoreType.DMA,
            pltpu.SemaphoreType.DMA,
            pltpu.SemaphoreType.DMA,
            pltpu.SemaphoreType.DMA,
        ],
        compiler_params=pltpu.CompilerParams(use_tc_tiling_on_sc=False,
                                             needs_layout_passes=False),
    )
    def k(tbl_ref, px_ref, py_ref, out_ref, pxb, pyb, idx4, w4, bufs, acc,
          gsem0, gsem1, osem0, osem1):
        gsem = (gsem0, gsem1)
        osem = (osem0, osem1)
        wid = lax.axis_index("s") * _NC + lax.axis_index("c")
        start = base * wid + jnp.minimum(wid, rem)
        n_w = base + (wid < rem).astype(jnp.int32)

        lane2 = lax.broadcasted_iota(jnp.int32, (_L,), 0) * 2
        pltpu.sync_copy(px_ref.at[pl.ds(start * _K, base * _K)],
                        pxb.at[pl.ds(0, base * _K)])
        pltpu.sync_copy(py_ref.at[pl.ds(start * _K, base * _K)],
                        pyb.at[pl.ds(0, base * _K)])

        @pl.when(n_w > base)
        def _():
            pltpu.sync_copy(px_ref.at[pl.ds((start + base) * _K, _K)],
                            pxb.at[pl.ds(base * _K, _K)])
            pltpu.sync_copy(py_ref.at[pl.ds((start + base) * _K, _K)],
                            pyb.at[pl.ds(base * _K, _K)])

        def stage_in(par, t):
            cid = start + t
            tb = (cid // cpb) * (Hx * Wx)
            for i in range(_K // _L):
                sl = pl.ds(i * _L, _L)
                bsl = pl.ds(t * _K + i * _L, _L)
                px = pxb[bsl]
                py = pyb[bsl]
                x0 = jnp.clip(px.astype(jnp.int32), 0, Wx - 1)
                x1 = jnp.minimum(x0 + 1, Wx - 1)
                y0 = jnp.clip(py.astype(jnp.int32), 0, Hx - 1)
                y1 = jnp.minimum(y0 + 1, Hx - 1)
                x0f = x0.astype(jnp.float32)
                x1f = x1.astype(jnp.float32)
                y0f = y0.astype(jnp.float32)
                y1f = y1.astype(jnp.float32)
                idx4[par, 0, sl] = tb + y0 * Wx + x0
                idx4[par, 1, sl] = tb + y1 * Wx + x0
                idx4[par, 2, sl] = tb + y0 * Wx + x1
                idx4[par, 3, sl] = tb + y1 * Wx + x1
                w4[par, 0, sl] = (x1f - px) * (y1f - py)
                w4[par, 1, sl] = (x1f - px) * (py - y0f)
                w4[par, 2, sl] = (px - x0f) * (y1f - py)
                w4[par, 3, sl] = (px - x0f) * (py - y0f)
            for q in range(4):
                pltpu.async_copy(tbl_ref.at[idx4.at[par, q]],
                                 bufs.at[par, q], gsem[par])

        def stage_out(par, t):
            cid = start + t
            b = cid // cpb
            n0 = (cid - b * cpb) * _K
            for _ in range(4):
                pltpu.make_async_copy(tbl_ref.at[idx4.at[par, 0]],
                                      bufs.at[par, 0], gsem[par]).wait()

            @pl.when(t >= 2)
            def _():
                cid2 = cid - 2
                b2 = cid2 // cpb
                n2 = (cid2 - b2 * cpb) * _K
                pltpu.make_async_copy(acc.at[par],
                                      out_ref.at[b2, pl.ds(n2, _K)],
                                      osem[par]).wait()

            def fma_body(g, _):
                gb = g * _L
                slg = pl.ds(gb, _L)
                wa16 = w4[par, 0, slg]
                wb16 = w4[par, 1, slg]
                wc16 = w4[par, 2, slg]
                wd16 = w4[par, 3, slg]
                for jj in range(_L):
                    j = gb + jj
                    ws = []
                    for w16 in (wa16, wb16, wc16, wd16):
                        wf = jnp.broadcast_to(w16[jj], (_L,))
                        ws.append(plsc.pack(
                            wf, wf, format=plsc.PackFormat.INTERLEAVED))
                    wav, wbv, wcv, wdv = ws
                    arow = acc.at[par, j]
                    for blk in range(C // (2 * _L)):
                        slb = pl.ds(blk * 2 * _L, 2 * _L)
                        s = ((wav * bufs[par, 0, j, slb]
                              + wbv * bufs[par, 1, j, slb])
                             + wcv * bufs[par, 2, j, slb]) \
                            + wdv * bufs[par, 3, j, slb]
                        lo, hi = plsc.unpack(
                            s, format=plsc.PackFormat.INTERLEAVED)
                        plsc.store_scatter(arow, [lane2 + blk * 2 * _L], lo)
                        plsc.store_scatter(arow, [lane2 + blk * 2 * _L + 1], hi)
                return 0

            lax.fori_loop(0, _K // _L, fma_body, 0)
            pltpu.async_copy(acc.at[par], out_ref.at[b, pl.ds(n0, _K)],
                             osem[par])

        stage_in(0, 0)

        def pair_body(tt, carry):
            for par in range(2):
                t = tt * 2 + par

                @pl.when(t < n_w)
                def _():
                    @pl.when(t + 1 < n_w)
                    def _():
                        stage_in(1 - par, t + 1)

                    stage_out(par, t)
            return carry

        lax.fori_loop(0, (maxw + 1) // 2, pair_body, 0)

        for par in range(2):
            tl = n_w - 1 - ((n_w - 1 + par) & 1)
            cidl = start + tl
            bl = cidl // cpb
            nl = (cidl - bl * cpb) * _K
            pltpu.make_async_copy(acc.at[par],
                                  out_ref.at[bl, pl.ds(nl, _K)],
                                  osem[par]).wait()

    return k(tbl, pxs, pys)


def kernel(x, pos, H, W):
    B, C, Hx, Wx = x.shape
    N = pos.shape[1]
    P = B * N
    tbl = x.astype(jnp.bfloat16).transpose(0, 2, 3, 1).reshape(B * Hx * Wx, C)
    pxs = pos[..., 0].reshape(P) * (Wx - 1) / W
    pys = pos[..., 1].reshape(P) * (Hx - 1) / H
    return _interp_sc(tbl, pxs, pys, B, C, Hx, Wx, N)


# speedup vs baseline: 1.1420x; 1.1420x over previous
import functools

import jax
import jax.numpy as jnp
from jax import lax
from jax.experimental import pallas as pl
from jax.experimental.pallas import tpu as pltpu
from jax.experimental.pallas import tpu_sc as plsc

_NC = 2
_NS = 16
_NW = _NC * _NS
_L = 16
_K = 80


def _interp_sc(tbl, pxs, pys, B, C, Hx, Wx, N):
    P = B * N
    n_chunks = P // _K
    cpb = N // _K
    base = n_chunks // _NW
    rem = n_chunks % _NW
    maxw = base + 1

    mesh = plsc.VectorSubcoreMesh(
        core_axis_name="c", subcore_axis_name="s",
        num_cores=_NC, num_subcores=_NS)

    @functools.partial(
        pl.kernel,
        out_type=jax.ShapeDtypeStruct((B, N, C), jnp.float32),
        mesh=mesh,
        scratch_types=[
            pltpu.VMEM((maxw * _K,), jnp.float32),
            pltpu.VMEM((maxw * _K,), jnp.float32),
            pltpu.VMEM((2, 2, _K), jnp.int32),
            pltpu.VMEM((2, 4, _K), jnp.float32),
            pltpu.VMEM((2, 2, _K, 2 * C), jnp.float32),
            pltpu.VMEM((2, _K, C), jnp.float32),
            pltpu.SemaphoreType.DMA,
            pltpu.SemaphoreType.DMA,
            pltpu.SemaphoreType.DMA,
            pltpu.SemaphoreType.DMA,
        ],
        compiler_params=pltpu.CompilerParams(use_tc_tiling_on_sc=True,
                                             needs_layout_passes=False),
    )
    def k(tbl_ref, px_ref, py_ref, out_ref, pxb, pyb, idx4, w4, bufs, acc,
          gsem0, gsem1, osem0, osem1):
        gsem = (gsem0, gsem1)
        osem = (osem0, osem1)
        wid = lax.axis_index("s") * _NC + lax.axis_index("c")
        start = base * wid + jnp.minimum(wid, rem)
        n_w = base + (wid < rem).astype(jnp.int32)

        pltpu.sync_copy(px_ref.at[pl.ds(start * _K, base * _K)],
                        pxb.at[pl.ds(0, base * _K)])
        pltpu.sync_copy(py_ref.at[pl.ds(start * _K, base * _K)],
                        pyb.at[pl.ds(0, base * _K)])

        @pl.when(n_w > base)
        def _():
            pltpu.sync_copy(px_ref.at[pl.ds((start + base) * _K, _K)],
                            pxb.at[pl.ds(base * _K, _K)])
            pltpu.sync_copy(py_ref.at[pl.ds((start + base) * _K, _K)],
                            pyb.at[pl.ds(base * _K, _K)])

        def stage_in(par, t):
            cid = start + t
            tb = (cid // cpb) * (Hx * Wx)
            for i in range(_K // _L):
                sl = pl.ds(i * _L, _L)
                bsl = pl.ds(t * _K + i * _L, _L)
                px = pxb[bsl]
                py = pyb[bsl]
                x0 = jnp.clip(px.astype(jnp.int32), 0, Wx - 1)
                x1 = jnp.minimum(x0 + 1, Wx - 1)
                y0 = jnp.clip(py.astype(jnp.int32), 0, Hx - 1)
                y1 = jnp.minimum(y0 + 1, Hx - 1)
                x0f = x0.astype(jnp.float32)
                x1f = x1.astype(jnp.float32)
                y0f = y0.astype(jnp.float32)
                y1f = y1.astype(jnp.float32)
                idx4[par, 0, sl] = tb + y0 * Wx + x0
                idx4[par, 1, sl] = tb + y0 * Wx + x1
                w4[par, 0, sl] = (x1f - px) * (y1f - py)
                w4[par, 1, sl] = (x1f - px) * (py - y0f)
                w4[par, 2, sl] = (px - x0f) * (y1f - py)
                w4[par, 3, sl] = (px - x0f) * (py - y0f)
            for q in range(2):
                pltpu.async_copy(tbl_ref.at[idx4.at[par, q]],
                                 bufs.at[par, q], gsem[par])

        def stage_out(par, t):
            cid = start + t
            b = cid // cpb
            n0 = (cid - b * cpb) * _K
            for _ in range(2):
                pltpu.make_async_copy(tbl_ref.at[idx4.at[par, 0]],
                                      bufs.at[par, 0], gsem[par]).wait()

            @pl.when(t >= 2)
            def _():
                cid2 = cid - 2
                b2 = cid2 // cpb
                n2 = (cid2 - b2 * cpb) * _K
                pltpu.make_async_copy(acc.at[par],
                                      out_ref.at[b2, pl.ds(n2, _K)],
                                      osem[par]).wait()

            def fma_body(g, _):
                gb = g * _L
                slg = pl.ds(gb, _L)
                wa16 = w4[par, 0, slg]
                wb16 = w4[par, 1, slg]
                wc16 = w4[par, 2, slg]
                wd16 = w4[par, 3, slg]
                for jj in range(_L):
                    j = gb + jj
                    wa = wa16[jj]
                    wb = wb16[jj]
                    wc = wc16[jj]
                    wd = wd16[jj]
                    for cg in range(C // _L):
                        slc = pl.ds(cg * _L, _L)
                        slh = pl.ds(C + cg * _L, _L)
                        acc[par, j, slc] = ((wa * bufs[par, 0, j, slc]
                                             + wb * bufs[par, 0, j, slh])
                                            + wc * bufs[par, 1, j, slc]) \
                            + wd * bufs[par, 1, j, slh]
                return 0

            lax.fori_loop(0, _K // _L, fma_body, 0)
            pltpu.async_copy(acc.at[par], out_ref.at[b, pl.ds(n0, _K)],
                             osem[par])

        stage_in(0, 0)

        def pair_body(tt, carry):
            for par in range(2):
                t = tt * 2 + par

                @pl.when(t < n_w)
                def _():
                    @pl.when(t + 1 < n_w)
                    def _():
                        stage_in(1 - par, t + 1)

                    stage_out(par, t)
            return carry

        lax.fori_loop(0, (maxw + 1) // 2, pair_body, 0)

        for par in range(2):
            tl = n_w - 1 - ((n_w - 1 + par) & 1)
            cidl = start + tl
            bl = cidl // cpb
            nl = (cidl - bl * cpb) * _K
            pltpu.make_async_copy(acc.at[par],
                                  out_ref.at[bl, pl.ds(nl, _K)],
                                  osem[par]).wait()

    return k(tbl, pxs, pys)


def kernel(x, pos, H, W):
    B, C, Hx, Wx = x.shape
    N = pos.shape[1]
    P = B * N
    xt = x.transpose(0, 2, 3, 1)
    xt_dn = jnp.concatenate([xt[:, 1:], xt[:, -1:]], axis=1)
    tbl = jnp.concatenate([xt, xt_dn], axis=-1).reshape(B * Hx * Wx, 2 * C)
    pxs = pos[..., 0].reshape(P) * (Wx - 1) / W
    pys = pos[..., 1].reshape(P) * (Hx - 1) / H
    return _interp_sc(tbl, pxs, pys, B, C, Hx, Wx, N)


# speedup vs baseline: 1.2312x; 1.0781x over previous
import functools

import jax
import jax.numpy as jnp
from jax import lax
from jax.experimental import pallas as pl
from jax.experimental.pallas import tpu as pltpu
from jax.experimental.pallas import tpu_sc as plsc

_NC = 2
_NS = 16
_NW = _NC * _NS
_L = 16
_K = 80


def _interp_sc(tbl, pxs, pys, B, C, Hx, Wx, N):
    P = B * N
    n_chunks = P // _K
    cpb = N // _K
    base = n_chunks // _NW
    rem = n_chunks % _NW
    maxw = base + 1

    mesh = plsc.VectorSubcoreMesh(
        core_axis_name="c", subcore_axis_name="s",
        num_cores=_NC, num_subcores=_NS)

    @functools.partial(
        pl.kernel,
        out_type=jax.ShapeDtypeStruct((P // 2, 2 * C), jnp.float32),
        mesh=mesh,
        scratch_types=[
            pltpu.VMEM((maxw * _K,), jnp.float32),
            pltpu.VMEM((maxw * _K,), jnp.float32),
            pltpu.VMEM((2, 4, _K), jnp.int32),
            pltpu.VMEM((2, 4, _K), jnp.float32),
            pltpu.VMEM((2, 4, _K, C), jnp.float32),
            pltpu.VMEM((2, _K // 2, 2 * C), jnp.float32),
            pltpu.SemaphoreType.DMA,
            pltpu.SemaphoreType.DMA,
            pltpu.SemaphoreType.DMA,
            pltpu.SemaphoreType.DMA,
        ],
        compiler_params=pltpu.CompilerParams(use_tc_tiling_on_sc=False,
                                             needs_layout_passes=False),
    )
    def k(tbl_ref, px_ref, py_ref, out_ref, pxb, pyb, idx4, w4, bufs, acc,
          gsem0, gsem1, osem0, osem1):
        gsem = (gsem0, gsem1)
        osem = (osem0, osem1)
        wid = lax.axis_index("s") * _NC + lax.axis_index("c")
        start = base * wid + jnp.minimum(wid, rem)
        n_w = base + (wid < rem).astype(jnp.int32)

        pltpu.sync_copy(px_ref.at[pl.ds(start * _K, base * _K)],
                        pxb.at[pl.ds(0, base * _K)])
        pltpu.sync_copy(py_ref.at[pl.ds(start * _K, base * _K)],
                        pyb.at[pl.ds(0, base * _K)])

        @pl.when(n_w > base)
        def _():
            pltpu.sync_copy(px_ref.at[pl.ds((start + base) * _K, _K)],
                            pxb.at[pl.ds(base * _K, _K)])
            pltpu.sync_copy(py_ref.at[pl.ds((start + base) * _K, _K)],
                            pyb.at[pl.ds(base * _K, _K)])

        def stage_in(par, t):
            cid = start + t
            tb = (cid // cpb) * (Hx * Wx)
            for i in range(_K // _L):
                sl = pl.ds(i * _L, _L)
                bsl = pl.ds(t * _K + i * _L, _L)
                px = pxb[bsl]
                py = pyb[bsl]
                x0 = jnp.clip(px.astype(jnp.int32), 0, Wx - 1)
                x1 = jnp.minimum(x0 + 1, Wx - 1)
                y0 = jnp.clip(py.astype(jnp.int32), 0, Hx - 1)
                y1 = jnp.minimum(y0 + 1, Hx - 1)
                x0f = x0.astype(jnp.float32)
                x1f = x1.astype(jnp.float32)
                y0f = y0.astype(jnp.float32)
                y1f = y1.astype(jnp.float32)
                idx4[par, 0, sl] = tb + y0 * Wx + x0
                idx4[par, 1, sl] = tb + y1 * Wx + x0
                idx4[par, 2, sl] = tb + y0 * Wx + x1
                idx4[par, 3, sl] = tb + y1 * Wx + x1
                w4[par, 0, sl] = (x1f - px) * (y1f - py)
                w4[par, 1, sl] = (x1f - px) * (py - y0f)
                w4[par, 2, sl] = (px - x0f) * (y1f - py)
                w4[par, 3, sl] = (px - x0f) * (py - y0f)
            for q in range(4):
                pltpu.async_copy(tbl_ref.at[idx4.at[par, q]],
                                 bufs.at[par, q], gsem[par])

        def stage_out(par, t):
            cid = start + t
            r0 = cid * (_K // 2)
            for _ in range(4):
                pltpu.make_async_copy(tbl_ref.at[idx4.at[par, 0]],
                                      bufs.at[par, 0], gsem[par]).wait()

            @pl.when(t >= 2)
            def _():
                r2 = (cid - 2) * (_K // 2)
                pltpu.make_async_copy(acc.at[par],
                                      out_ref.at[pl.ds(r2, _K // 2)],
                                      osem[par]).wait()

            def fma_body(g, _):
                gb = g * _L
                slg = pl.ds(gb, _L)
                wa16 = w4[par, 0, slg]
                wb16 = w4[par, 1, slg]
                wc16 = w4[par, 2, slg]
                wd16 = w4[par, 3, slg]
                for jj in range(_L):
                    j = gb + jj
                    ar = g * (_L // 2) + jj // 2
                    ac = (jj % 2) * C
                    wa = wa16[jj]
                    wb = wb16[jj]
                    wc = wc16[jj]
                    wd = wd16[jj]
                    for cg in range(C // _L):
                        slc = pl.ds(cg * _L, _L)
                        acc[par, ar, pl.ds(ac + cg * _L, _L)] = (
                            (wa * bufs[par, 0, j, slc]
                             + wb * bufs[par, 1, j, slc])
                            + wc * bufs[par, 2, j, slc]) \
                            + wd * bufs[par, 3, j, slc]
                return 0

            lax.fori_loop(0, _K // _L, fma_body, 0)
            pltpu.async_copy(acc.at[par], out_ref.at[pl.ds(r0, _K // 2)],
                             osem[par])

        stage_in(0, 0)

        def pair_body(tt, carry):
            for par in range(2):
                t = tt * 2 + par

                @pl.when(t < n_w)
                def _():
                    @pl.when(t + 1 < n_w)
                    def _():
                        stage_in(1 - par, t + 1)

                    stage_out(par, t)
            return carry

        lax.fori_loop(0, (maxw + 1) // 2, pair_body, 0)

        for par in range(2):
            tl = n_w - 1 - ((n_w - 1 + par) & 1)
            rl = (start + tl) * (_K // 2)
            pltpu.make_async_copy(acc.at[par],
                                  out_ref.at[pl.ds(rl, _K // 2)],
                                  osem[par]).wait()

    return k(tbl, pxs, pys)


def kernel(x, pos, H, W):
    B, C, Hx, Wx = x.shape
    N = pos.shape[1]
    P = B * N
    tbl = x.transpose(0, 2, 3, 1).reshape(B * Hx * Wx, C)
    pxs = pos[..., 0].reshape(P) * (Wx - 1) / W
    pys = pos[..., 1].reshape(P) * (Hx - 1) / H
    out = _interp_sc(tbl, pxs, pys, B, C, Hx, Wx, N)
    return out.reshape(B, N, C)


# speedup vs baseline: 1.2316x; 1.0004x over previous
import functools

import jax
import jax.numpy as jnp
from jax import lax
from jax.experimental import pallas as pl
from jax.experimental.pallas import tpu as pltpu
from jax.experimental.pallas import tpu_sc as plsc

_NC = 2
_NS = 16
_NW = _NC * _NS
_L = 16
_K = 80


def _interp_sc(tbl, pxs, pys, B, C, Hx, Wx, N):
    P = B * N
    n_chunks = P // _K
    cpb = N // _K
    base = n_chunks // _NW
    rem = n_chunks % _NW
    maxw = base + 1

    mesh = plsc.VectorSubcoreMesh(
        core_axis_name="c", subcore_axis_name="s",
        num_cores=_NC, num_subcores=_NS)

    @functools.partial(
        pl.kernel,
        out_type=jax.ShapeDtypeStruct((B, N, C), jnp.float32),
        mesh=mesh,
        scratch_types=[
            pltpu.VMEM((maxw * _K,), jnp.float32),
            pltpu.VMEM((maxw * _K,), jnp.float32),
            pltpu.VMEM((2, 4, _K), jnp.int32),
            pltpu.VMEM((2, 4, _K), jnp.float32),
            pltpu.VMEM((2, 4, _K, C), jnp.float32),
            pltpu.VMEM((2, _K, C), jnp.float32),
            pltpu.SemaphoreType.DMA,
            pltpu.SemaphoreType.DMA,
            pltpu.SemaphoreType.DMA,
            pltpu.SemaphoreType.DMA,
        ],
        compiler_params=pltpu.CompilerParams(use_tc_tiling_on_sc=False,
                                             needs_layout_passes=False),
    )
    def k(tbl_ref, px_ref, py_ref, out_ref, pxb, pyb, idx4, w4, bufs, acc,
          gsem0, gsem1, osem0, osem1):
        gsem = (gsem0, gsem1)
        osem = (osem0, osem1)
        wid = lax.axis_index("s") * _NC + lax.axis_index("c")
        start = base * wid + jnp.minimum(wid, rem)
        n_w = base + (wid < rem).astype(jnp.int32)

        pltpu.sync_copy(px_ref.at[pl.ds(start * _K, base * _K)],
                        pxb.at[pl.ds(0, base * _K)])
        pltpu.sync_copy(py_ref.at[pl.ds(start * _K, base * _K)],
                        pyb.at[pl.ds(0, base * _K)])

        @pl.when(n_w > base)
        def _():
            pltpu.sync_copy(px_ref.at[pl.ds((start + base) * _K, _K)],
                            pxb.at[pl.ds(base * _K, _K)])
            pltpu.sync_copy(py_ref.at[pl.ds((start + base) * _K, _K)],
                            pyb.at[pl.ds(base * _K, _K)])

        def stage_in(par, t):
            cid = start + t
            tb = (cid // cpb) * (Hx * Wx)
            for i in range(_K // _L):
                sl = pl.ds(i * _L, _L)
                bsl = pl.ds(t * _K + i * _L, _L)
                px = pxb[bsl]
                py = pyb[bsl]
                x0 = jnp.clip(px.astype(jnp.int32), 0, Wx - 1)
                x1 = jnp.minimum(x0 + 1, Wx - 1)
                y0 = jnp.clip(py.astype(jnp.int32), 0, Hx - 1)
                y1 = jnp.minimum(y0 + 1, Hx - 1)
                x0f = x0.astype(jnp.float32)
                x1f = x1.astype(jnp.float32)
                y0f = y0.astype(jnp.float32)
                y1f = y1.astype(jnp.float32)
                idx4[par, 0, sl] = tb + y0 * Wx + x0
                idx4[par, 1, sl] = tb + y1 * Wx + x0
                idx4[par, 2, sl] = tb + y0 * Wx + x1
                idx4[par, 3, sl] = tb + y1 * Wx + x1
                w4[par, 0, sl] = (x1f - px) * (y1f - py)
                w4[par, 1, sl] = (x1f - px) * (py - y0f)
                w4[par, 2, sl] = (px - x0f) * (y1f - py)
                w4[par, 3, sl] = (px - x0f) * (py - y0f)
            for q in range(4):
                pltpu.async_copy(tbl_ref.at[idx4.at[par, q]],
                                 bufs.at[par, q], gsem[par])

        def stage_out(par, t):
            cid = start + t
            b = cid // cpb
            n0 = (cid - b * cpb) * _K
            for _ in range(4):
                pltpu.make_async_copy(tbl_ref.at[idx4.at[par, 0]],
                                      bufs.at[par, 0], gsem[par]).wait()

            @pl.when(t >= 2)
            def _():
                cid2 = cid - 2
                b2 = cid2 // cpb
                n2 = (cid2 - b2 * cpb) * _K
                pltpu.make_async_copy(acc.at[par],
                                      out_ref.at[b2, pl.ds(n2, _K)],
                                      osem[par]).wait()

            def fma_body(g, _):
                gb = g * _L
                slg = pl.ds(gb, _L)
                wa16 = w4[par, 0, slg]
                wb16 = w4[par, 1, slg]
                wc16 = w4[par, 2, slg]
                wd16 = w4[par, 3, slg]
                for jj in range(_L):
                    j = gb + jj
                    wa = wa16[jj]
                    wb = wb16[jj]
                    wc = wc16[jj]
                    wd = wd16[jj]
                    for cg in range(C // _L):
                        slc = pl.ds(cg * _L, _L)
                        acc[par, j, slc] = ((wa * bufs[par, 0, j, slc]
                                             + wb * bufs[par, 1, j, slc])
                                            + wc * bufs[par, 2, j, slc]) \
                            + wd * bufs[par, 3, j, slc]
                return 0

            lax.fori_loop(0, _K // _L, fma_body, 0)
            pltpu.async_copy(acc.at[par], out_ref.at[b, pl.ds(n0, _K)],
                             osem[par])

        stage_in(0, 0)

        def pair_body(tt, carry):
            for par in range(2):
                t = tt * 2 + par

                @pl.when(t < n_w)
                def _():
                    @pl.when(t + 1 < n_w)
                    def _():
                        stage_in(1 - par, t + 1)

                    stage_out(par, t)
            return carry

        lax.fori_loop(0, (maxw + 1) // 2, pair_body, 0)

        for par in range(2):
            tl = n_w - 1 - ((n_w - 1 + par) & 1)
            cidl = start + tl
            bl = cidl // cpb
            nl = (cidl - bl * cpb) * _K
            pltpu.make_async_copy(acc.at[par],
                                  out_ref.at[bl, pl.ds(nl, _K)],
                                  osem[par]).wait()

    return k(tbl, pxs, pys)


def kernel(x, pos, H, W):
    B, C, Hx, Wx = x.shape
    N = pos.shape[1]
    P = B * N
    tbl = x.transpose(0, 2, 3, 1).reshape(B * Hx * Wx, C)
    pxs = pos[..., 0].reshape(P) * (Wx - 1) / W
    pys = pos[..., 1].reshape(P) * (Hx - 1) / H
    return _interp_sc(tbl, pxs, pys, B, C, Hx, Wx, N)
